# baseline (speedup 1.0000x reference)
import functools

import jax
import jax.numpy as jnp
from jax import lax
from jax.experimental import pallas as pl
from jax.experimental.pallas import tpu as pltpu
from jax.experimental.pallas import tpu_sc as plsc

BATCH = 4096
HIST = 200
D = 64
NC = 2
NS = 16
NW = NC * NS
BPW = BATCH // NW

_mesh = plsc.VectorSubcoreMesh(core_axis_name="c", subcore_axis_name="s")


@functools.partial(
    pl.kernel,
    out_type=jax.ShapeDtypeStruct((BATCH, HIST, D), jnp.float32),
    mesh=_mesh,
    scratch_types=[
        pltpu.VMEM((BPW, HIST), jnp.int32),
        pltpu.VMEM((2, HIST, D), jnp.float32),
        pltpu.SemaphoreType.DMA,
    ],
    compiler_params=pltpu.CompilerParams(use_tc_tiling_on_sc=False),
)
def _sc_gather(idx_hbm, table_hbm, out_hbm, idx_v, rows_v, gsem):
    wid = lax.axis_index("s") * NC + lax.axis_index("c")
    base = wid * BPW
    pltpu.sync_copy(idx_hbm.at[pl.ds(base, BPW), :], idx_v)

    def start_gather(r, slot):
        pltpu.async_copy(
            table_hbm.at[idx_v.at[r]],
            rows_v.at[slot],
            gsem,
        )

    def wait_gather(slot):
        pltpu.make_async_copy(
            table_hbm.at[pl.ds(0, HIST)], rows_v.at[slot], gsem
        ).wait()

    start_gather(0, 0)

    def body(r, _):
        slot = lax.rem(r, 2)
        nslot = lax.rem(r + 1, 2)

        @pl.when(r + 1 < BPW)
        def _():
            start_gather(r + 1, nslot)

        wait_gather(slot)
        pltpu.sync_copy(rows_v.at[slot], out_hbm.at[base + r])
        return 0

    lax.fori_loop(0, BPW, body, 0)


def kernel(indices, embed_weight):
    return _sc_gather(indices.astype(jnp.int32), embed_weight)


# speedup vs baseline: 1.2261x; 1.2261x over previous
import functools

import jax
import jax.numpy as jnp
from jax import lax
from jax.experimental import pallas as pl
from jax.experimental.pallas import tpu as pltpu
from jax.experimental.pallas import tpu_sc as plsc

BATCH = 4096
HIST = 200
D = 64
DP = 128
NC = 2
NS = 16
NW = NC * NS
BPW = BATCH // NW

_mesh = plsc.VectorSubcoreMesh(core_axis_name="c", subcore_axis_name="s")


@functools.partial(
    pl.kernel,
    out_type=jax.ShapeDtypeStruct((BATCH, HIST, DP), jnp.float32),
    mesh=_mesh,
    scratch_types=[
        pltpu.VMEM((HIST, BPW), jnp.int32),
        pltpu.VMEM((2, BPW, DP), jnp.float32),
        pltpu.SemaphoreType.DMA,
    ],
)
def _sc_gather(idxT_hbm, tab_hbm, out_hbm, idx_v, rows_v, gsem):
    wid = lax.axis_index("s") * NC + lax.axis_index("c")
    b0 = wid * BPW
    pltpu.sync_copy(idxT_hbm.at[:, pl.ds(b0, BPW)], idx_v)

    def start_gather(h, slot):
        pltpu.async_copy(
            tab_hbm.at[idx_v.at[h]],
            rows_v.at[slot],
            gsem,
        )

    def wait_gather(slot):
        pltpu.make_async_copy(
            tab_hbm.at[pl.ds(0, BPW)], rows_v.at[slot], gsem
        ).wait()

    start_gather(0, 0)

    def body(h, _):
        slot = lax.rem(h, 2)
        nslot = lax.rem(h + 1, 2)

        @pl.when(h + 1 < HIST)
        def _():
            start_gather(h + 1, nslot)

        wait_gather(slot)
        pltpu.sync_copy(
            rows_v.at[slot],
            out_hbm.at[pl.ds(b0, BPW), h, :],
        )
        return 0

    lax.fori_loop(0, HIST, body, 0)


def kernel(indices, embed_weight):
    idx_t = jnp.transpose(indices.astype(jnp.int32))
    tab128 = jnp.pad(embed_weight, ((0, 0), (0, DP - D)))
    return _sc_gather(idx_t, tab128)[:, :, :D]
